# baseline (speedup 1.0000x reference)
import functools

import jax
import jax.numpy as jnp
from jax import lax
from jax.experimental import pallas as pl
from jax.experimental.pallas import tpu as pltpu
from jax.experimental.pallas import tpu_sc as plsc

BATCH = 4096
HIST = 50
DIM = 64
LANES = 16
NC = 2
NS = 16
NW = NC * NS
PER_W = BATCH // NW
DREG = DIM // LANES
GROUPS = PER_W // LANES
INV_HIST = 1.0 / HIST


def _emb_body(tokens_hbm, table_hbm, out_hbm, idx_raw, idx_t, acc_v,
              sem, idx_sem):
    wid = lax.axis_index("s") * NC + lax.axis_index("c")
    base = wid * PER_W
    idx_cp = pltpu.async_copy(tokens_hbm.at[pl.ds(base, PER_W)], idx_raw,
                              idx_sem)

    zeros = jnp.zeros((LANES,), jnp.float32)

    def zero_elem(e, carry):
        for d in range(DREG):
            acc_v[e, pl.ds(d * LANES, LANES)] = zeros
        return carry

    lax.fori_loop(0, PER_W, zero_elem, 0)
    idx_cp.wait()

    lane = lax.iota(jnp.int32, LANES)

    def transpose_t(t, carry):
        col = jnp.full((LANES,), 0, jnp.int32) + t
        for g in range(GROUPS):
            rows = lane + (g * LANES)
            idx_t[t, pl.ds(g * LANES, LANES)] = plsc.load_gather(
                idx_raw, [rows, col])
        return carry

    lax.fori_loop(0, HIST, transpose_t, 0)

    def fire(t, carry):
        pltpu.async_copy(table_hbm.at[idx_t.at[t]], acc_v, sem, add=True)
        return carry

    lax.fori_loop(0, HIST, fire, 0)

    def drain(t, carry):
        pltpu.make_async_copy(table_hbm.at[idx_t.at[t]], acc_v, sem).wait()
        return carry

    lax.fori_loop(0, HIST, drain, 0)

    def scale(e, carry):
        for d in range(DREG):
            sl = pl.ds(d * LANES, LANES)
            acc_v[e, sl] = acc_v[e, sl] * INV_HIST
        return carry

    lax.fori_loop(0, PER_W, scale, 0)
    pltpu.sync_copy(acc_v, out_hbm.at[pl.ds(base, PER_W)])


@functools.partial(
    pl.kernel,
    out_type=jax.ShapeDtypeStruct((BATCH, DIM), jnp.float32),
    mesh=plsc.VectorSubcoreMesh(core_axis_name="c", subcore_axis_name="s"),
    scratch_types=[
        pltpu.VMEM((PER_W, HIST), jnp.int32),
        pltpu.VMEM((HIST, PER_W), jnp.int32),
        pltpu.VMEM((PER_W, DIM), jnp.float32),
        pltpu.SemaphoreType.DMA,
        pltpu.SemaphoreType.DMA,
    ],
    compiler_params=pltpu.CompilerParams(use_tc_tiling_on_sc=False,
                                         needs_layout_passes=False),
)
def _emb(tokens_hbm, table_hbm, out_hbm, idx_raw, idx_t, acc_v, sem, idx_sem):
    _emb_body(tokens_hbm, table_hbm, out_hbm, idx_raw, idx_t, acc_v,
              sem, idx_sem)


def kernel(tokens_list, table):
    return _emb(tokens_list, table)
